# baseline (speedup 1.0000x reference)
import functools

import jax
import jax.numpy as jnp
from jax import lax
from jax.experimental import pallas as pl
from jax.experimental.pallas import tpu as pltpu
from jax.experimental.pallas import tpu_sc as plsc

B, T, F = 1024, 200, 128
EMB = 64
OUTW = (F - 1) + EMB
NC, NS, L = 2, 16, 16
NW = NC * NS
TS_T, TS_B = 8, 128
NST = (T // TS_T) * (B // TS_B)
SPW = (NST + NW - 1) // NW


def _ids_body(x_ref, ids_ref):
    v = x_ref[:, :, 0]
    ids_ref[...] = (v - 1.0).astype(jnp.int32).T


def _ids_kernel(x):
    bb = 128
    return pl.pallas_call(
        _ids_body,
        grid=(B // bb,),
        in_specs=[pl.BlockSpec((bb, T, F), lambda i: (i, 0, 0))],
        out_specs=pl.BlockSpec((T, bb), lambda i: (0, i)),
        out_shape=jax.ShapeDtypeStruct((T, B), jnp.int32),
    )(x)


def _sc_body(table_hbm, ids_hbm, out_hbm,
             idxv, gb0, gb1, stg, g0, g1, ssem):
    cid = lax.axis_index("c")
    sid = lax.axis_index("s")
    wid = sid * NC + cid

    gbufs = (gb0, gb1)
    gsems = (g0, g1)
    iota16 = lax.iota(jnp.int32, L)
    zeros16 = jnp.zeros((L,), jnp.int32)

    def gath(tt, gb):
        return pltpu.make_async_copy(
            table_hbm.at[idxv.at[tt]], gbufs[gb], gsems[gb])

    def transpose_tt(tt, gb):
        gbuf = gbufs[gb]

        @plsc.parallel_loop(0, EMB, step=1, unroll=4)
        def _feat(e):
            esplat = zeros16 + e
            vs = [plsc.load_gather(gbuf, [iota16 + k * L, esplat])
                  for k in range(TS_B // L)]
            for k in range(TS_B // L):
                stg[e, tt, pl.ds(k * L, L)] = vs[k]

    def stg_copy(t0, b0):
        return pltpu.make_async_copy(
            stg,
            out_hbm.at[pl.ds(F - 1, EMB), pl.ds(t0, TS_T), pl.ds(b0, TS_B)],
            ssem)

    def supertile(s, carry):
        st = wid + NW * s

        @pl.when(st < NST)
        def _():
            t0 = (st // (B // TS_B)) * TS_T
            b0 = (st % (B // TS_B)) * TS_B
            pltpu.sync_copy(
                ids_hbm.at[pl.ds(t0, TS_T), pl.ds(b0, TS_B)], idxv)
            gath(0, 0).start()
            for tt in range(TS_T):
                gb = tt % 2
                gath(tt, gb).wait()
                if tt + 1 < TS_T:
                    gath(tt + 1, 1 - gb).start()
                transpose_tt(tt, gb)
            stg_copy(t0, b0).start()
            stg_copy(t0, b0).wait()
        return carry

    lax.fori_loop(0, SPW, supertile, 0)


def _sc_kernel(table128, ids_t):
    mesh = plsc.VectorSubcoreMesh(core_axis_name="c", subcore_axis_name="s")
    return pl.kernel(
        _sc_body,
        mesh=mesh,
        compiler_params=pltpu.CompilerParams(needs_layout_passes=False),
        out_type=jax.ShapeDtypeStruct((OUTW, T, B), jnp.float32),
        scratch_types=[
            pltpu.VMEM((TS_T, TS_B), jnp.int32),
            pltpu.VMEM((TS_B, F), jnp.float32),
            pltpu.VMEM((TS_B, F), jnp.float32),
            pltpu.VMEM((EMB, TS_T, TS_B), jnp.float32),
            pltpu.SemaphoreType.DMA,
            pltpu.SemaphoreType.DMA,
            pltpu.SemaphoreType.DMA,
        ],
    )(table128, ids_t)


def _xpose_body(x_ref, outt_in_ref, out_ref):
    del outt_in_ref
    for tt in range(x_ref.shape[1]):
        xt = x_ref[:, tt, :].T
        out_ref[:, tt, :] = xt[1:, :]


def _xpose_kernel(x, out_t):
    tb, bb = 40, 256
    return pl.pallas_call(
        _xpose_body,
        grid=(T // tb, B // bb),
        in_specs=[
            pl.BlockSpec((bb, tb, F), lambda t, b: (b, t, 0)),
            pl.BlockSpec(memory_space=pltpu.MemorySpace.HBM),
        ],
        out_specs=pl.BlockSpec((F - 1, tb, bb), lambda t, b: (0, t, b)),
        out_shape=jax.ShapeDtypeStruct((OUTW, T, B), jnp.float32),
        input_output_aliases={1: 0},
    )(x, out_t)


@jax.jit
def _run(x, table):
    table128 = jnp.pad(table, ((0, 0), (0, F - EMB)))
    ids_t = _ids_kernel(x)
    out_t = _sc_kernel(table128, ids_t)
    out_t = _xpose_kernel(x, out_t)
    return jnp.transpose(out_t, (2, 1, 0))


def kernel(x, table):
    return _run(x, table)
